# baseline (speedup 1.0000x reference)
import functools

import jax
import jax.numpy as jnp
from jax import lax
from jax.experimental import pallas as pl
from jax.experimental.pallas import tpu as pltpu
from jax.experimental.pallas import tpu_sc as plsc

_N = 4096
_K = 20
_C = 3
_B = 4
_R = 256

_NEG = -3.0e38

_NSUB = 32
_PLANE = _N * _K


def _topk_body(bofs, x_ref, xt_ref, widx_ref, ctr_ref):
    b = pl.program_id(0) + bofs
    xr = [x_ref[0, c : c + 1, :] for c in range(_C)]
    cc = [xt_ref[0, :, c : c + 1] for c in range(_C)]

    xsq = xr[0] * xr[0] + xr[1] * xr[1] + xr[2] * xr[2]
    csq = cc[0] * cc[0] + cc[1] * cc[1] + cc[2] * cc[2]

    xrb = [v.astype(jnp.bfloat16).astype(jnp.float32) for v in xr]
    ccb = [
        2.0 * v.astype(jnp.bfloat16).astype(jnp.float32) for v in cc
    ]
    dot2 = ccb[0] * xrb[0] + ccb[1] * xrb[1] + ccb[2] * xrb[2]
    work = dot2 - csq - xsq

    iota = lax.broadcasted_iota(jnp.int32, (_R, _N), 1)
    cols = []
    for kk in range(_K):
        m = jnp.max(work, axis=1, keepdims=True)
        cand = jnp.where(work == m, iota, _N)
        idx = jnp.min(cand, axis=1, keepdims=True)
        if kk < _K - 1:
            work = jnp.where(iota == idx, _NEG, work)
        cols.append(idx)
    idxm = jnp.concatenate(cols, axis=1)
    nb_base = 3 * (idxm + b * _N)
    for c in range(_C):
        widx_ref[c, 0] = nb_base + c
        ctr_ref[0, c] = jnp.broadcast_to(cc[c], (_R, _K))


def _topk_windices(x, xt, bofs):
    nb = x.shape[0]
    grid = (nb, _N // _R)
    return pl.pallas_call(
        functools.partial(_topk_body, bofs),
        grid=grid,
        in_specs=[
            pl.BlockSpec((1, _C, _N), lambda b, r: (b, 0, 0)),
            pl.BlockSpec((1, _R, _C), lambda b, r: (b, r, 0)),
        ],
        out_specs=[
            pl.BlockSpec((_C, 1, _R, _K), lambda b, r: (0, b, r, 0)),
            pl.BlockSpec((1, _C, _R, _K), lambda b, r: (b, 0, r, 0)),
        ],
        out_shape=[
            jax.ShapeDtypeStruct((_C, nb, _N, _K), jnp.int32),
            jax.ShapeDtypeStruct((nb, _C, _N, _K), jnp.float32),
        ],
        compiler_params=pltpu.CompilerParams(
            dimension_semantics=("parallel", "parallel"),
        ),
    )(x, xt)


def _gather_assemble(widx, ctr_flat, table, nb):
    p_total = nb * _N * _K
    pw = p_total // _NSUB
    chp = pw
    nchunk = 1
    grp = chp // 128
    mesh = plsc.VectorSubcoreMesh(core_axis_name="c", subcore_axis_name="s")

    @functools.partial(
        pl.kernel,
        mesh=mesh,
        out_type=jax.ShapeDtypeStruct((nb * _C * _PLANE,), jnp.float32),
        scratch_types=[
            pltpu.VMEM((_C * grp, 128), jnp.int32),
            pltpu.VMEM((2 * _C, chp), jnp.float32),
            pltpu.SemaphoreType.DMA,
        ],
    )
    def sck(widx_hbm, ctr_hbm, tab_hbm, out_hbm, widx_v, stage_v, sem):
        wid = lax.axis_index("s") * 2 + lax.axis_index("c")
        for chunk in range(nchunk):
            p0 = wid * pw + chunk * chp
            g0 = lax.div(p0, 128)
            b = lax.div(p0, _PLANE)
            local0 = p0 - b * _PLANE
            for c in range(_C):
                pltpu.sync_copy(
                    widx_hbm.at[c, pl.ds(g0, grp), :],
                    widx_v.at[pl.ds(c * grp, grp)],
                )

            def fire(g, carry):
                for c in range(_C):
                    pltpu.async_copy(
                        tab_hbm.at[widx_v.at[c * grp + g]],
                        stage_v.at[c, pl.ds(g * 128, 128)],
                        sem,
                    )
                return carry

            lax.fori_loop(0, grp, fire, 0)
            for c in range(_C):
                coff = (b * _C + c) * _PLANE + local0
                pltpu.sync_copy(
                    ctr_hbm.at[pl.ds(pl.multiple_of(coff, 8), chp)],
                    stage_v.at[_C + c],
                )
            for c in range(_C):
                pltpu.make_async_copy(
                    tab_hbm.at[pl.ds(0, chp)], stage_v.at[c], sem
                ).wait()

            def diff(i, carry):
                sl = pl.ds(i * 16, 16)
                for c in range(_C):
                    stage_v[c, sl] = stage_v[c, sl] - stage_v[_C + c, sl]
                return carry

            lax.fori_loop(0, chp // 16, diff, 0)
            for c in range(_C):
                off = (b * _C + c) * _PLANE + local0
                pltpu.sync_copy(
                    stage_v.at[c],
                    out_hbm.at[pl.ds(pl.multiple_of(off, 8), chp)],
                )

    return sck(widx, ctr_flat, table)


def kernel(x, k):
    del k
    xt = jnp.transpose(x, (0, 2, 1))
    table = xt.reshape(-1)
    widx, ctr = _topk_windices(x, xt, 0)
    diffp = _gather_assemble(
        widx.reshape(_C, _B * _N * _K // 128, 128),
        ctr.reshape(-1),
        table,
        _B,
    )
    return jnp.concatenate(
        [diffp.reshape(_B, _C, _N, _K), ctr], axis=1
    )


# speedup vs baseline: 1.0089x; 1.0089x over previous
import functools

import jax
import jax.numpy as jnp
from jax import lax
from jax.experimental import pallas as pl
from jax.experimental.pallas import tpu as pltpu
from jax.experimental.pallas import tpu_sc as plsc

_N = 4096
_K = 20
_C = 3
_B = 4
_R = 256

_NEG = -3.0e38

_NSUB = 32
_PLANE = _N * _K


def _topk_body(bofs, x_ref, xt_ref, widx_ref, ctr_ref):
    b = pl.program_id(0) + bofs
    xr = [x_ref[0, c : c + 1, :] for c in range(_C)]
    cc = [xt_ref[0, :, c : c + 1] for c in range(_C)]

    xsq = xr[0] * xr[0] + xr[1] * xr[1] + xr[2] * xr[2]
    csq = cc[0] * cc[0] + cc[1] * cc[1] + cc[2] * cc[2]

    xrb = [v.astype(jnp.bfloat16).astype(jnp.float32) for v in xr]
    ccb = [
        2.0 * v.astype(jnp.bfloat16).astype(jnp.float32) for v in cc
    ]
    dot2 = ccb[0] * xrb[0] + ccb[1] * xrb[1] + ccb[2] * xrb[2]
    work = dot2 - csq - xsq

    iota = lax.broadcasted_iota(jnp.int32, (_R, _N), 1)
    cols = []
    for kk in range(_K):
        m = jnp.max(work, axis=1, keepdims=True)
        cand = jnp.where(work == m, iota, _N)
        idx = jnp.min(cand, axis=1, keepdims=True)
        if kk < _K - 1:
            work = jnp.where(iota == idx, _NEG, work)
        cols.append(idx)
    idxm = jnp.concatenate(cols, axis=1)
    nb_base = 3 * (idxm + b * _N)
    for c in range(_C):
        widx_ref[c, 0] = nb_base + c
        ctr_ref[0, c] = jnp.broadcast_to(cc[c], (_R, _K))


def _topk_windices(x, xt, bofs):
    nb = x.shape[0]
    grid = (nb, _N // _R)
    return pl.pallas_call(
        functools.partial(_topk_body, bofs),
        grid=grid,
        in_specs=[
            pl.BlockSpec((1, _C, _N), lambda b, r: (b, 0, 0)),
            pl.BlockSpec((1, _R, _C), lambda b, r: (b, r, 0)),
        ],
        out_specs=[
            pl.BlockSpec((_C, 1, _R, _K), lambda b, r: (0, b, r, 0)),
            pl.BlockSpec((1, _C, _R, _K), lambda b, r: (b, 0, r, 0)),
        ],
        out_shape=[
            jax.ShapeDtypeStruct((_C, nb, _N, _K), jnp.int32),
            jax.ShapeDtypeStruct((nb, _C, _N, _K), jnp.float32),
        ],
        compiler_params=pltpu.CompilerParams(
            dimension_semantics=("parallel", "parallel"),
        ),
    )(x, xt)


def _gather_assemble(widx, ctr_flat, table, nb):
    p_total = nb * _N * _K
    pw = p_total // _NSUB
    chp = pw
    nchunk = 1
    grp = chp // 128
    mesh = plsc.VectorSubcoreMesh(core_axis_name="c", subcore_axis_name="s")

    @functools.partial(
        pl.kernel,
        mesh=mesh,
        out_type=jax.ShapeDtypeStruct((nb * _C * _PLANE,), jnp.float32),
        scratch_types=[
            pltpu.VMEM((_C * grp, 128), jnp.int32),
            pltpu.VMEM((2 * _C, chp), jnp.float32),
            pltpu.SemaphoreType.DMA,
        ],
    )
    def sck(widx_hbm, ctr_hbm, tab_hbm, out_hbm, widx_v, stage_v, sem):
        wid = lax.axis_index("s") * 2 + lax.axis_index("c")
        for chunk in range(nchunk):
            p0 = wid * pw + chunk * chp
            g0 = lax.div(p0, 128)
            b = lax.div(p0, _PLANE)
            local0 = p0 - b * _PLANE
            for c in range(_C):
                pltpu.sync_copy(
                    widx_hbm.at[c, pl.ds(g0, grp), :],
                    widx_v.at[pl.ds(c * grp, grp)],
                )

            def fire(g, carry):
                for c in range(_C):
                    pltpu.async_copy(
                        tab_hbm.at[widx_v.at[c * grp + g]],
                        stage_v.at[c, pl.ds(g * 128, 128)],
                        sem,
                    )
                return carry

            lax.fori_loop(0, grp, fire, 0)
            for c in range(_C):
                coff = (b * _C + c) * _PLANE + local0
                pltpu.sync_copy(
                    ctr_hbm.at[pl.ds(pl.multiple_of(coff, 8), chp)],
                    stage_v.at[_C + c],
                )
            for c in range(_C):
                pltpu.make_async_copy(
                    tab_hbm.at[pl.ds(0, chp)], stage_v.at[c], sem
                ).wait()

            def diff(i, carry):
                sl = pl.ds(i * 16, 16)
                for c in range(_C):
                    stage_v[c, sl] = stage_v[c, sl] - stage_v[_C + c, sl]
                return carry

            lax.fori_loop(0, chp // 16, diff, 0)
            for c in range(_C):
                off = (b * _C + c) * _PLANE + local0
                pltpu.sync_copy(
                    stage_v.at[c],
                    out_hbm.at[pl.ds(pl.multiple_of(off, 8), chp)],
                )

    return sck(widx, ctr_flat, table)


def kernel(x, k):
    del k
    xt = jnp.transpose(x, (0, 2, 1))
    table = xt.reshape(-1)
    halves = []
    hb = _B // 2
    for h in range(2):
        xs = x[h * hb : (h + 1) * hb]
        xts = xt[h * hb : (h + 1) * hb]
        widx, ctr = _topk_windices(xs, xts, h * hb)
        diffp = _gather_assemble(
            widx.reshape(_C, hb * _N * _K // 128, 128),
            ctr.reshape(-1),
            table,
            hb,
        )
        halves.append(
            jnp.concatenate([diffp.reshape(hb, _C, _N, _K), ctr], axis=1)
        )
    return jnp.concatenate(halves, axis=0)
